# baseline (speedup 1.0000x reference)
import functools

import jax
import jax.numpy as jnp
from jax import lax
from jax.experimental import pallas as pl
from jax.experimental.pallas import tpu as pltpu
from jax.experimental.pallas import tpu_sc as plsc

BATCH = 16384
EMB = 64
NC, NS = 2, 16
NW = NC * NS
BPW = BATCH // NW
CHUNK = 128
NCH = BPW // CHUNK

_mesh = plsc.VectorSubcoreMesh(core_axis_name="c", subcore_axis_name="s")


@functools.partial(
    pl.kernel,
    mesh=_mesh,
    out_type=jax.ShapeDtypeStruct((BATCH, 2 * EMB), jnp.float32),
    scratch_types=[
        pltpu.VMEM((NCH, CHUNK), jnp.int32),
        pltpu.VMEM((CHUNK, EMB), jnp.float32),
        pltpu.VMEM((CHUNK, EMB), jnp.float32),
        pltpu.VMEM((CHUNK, 2 * EMB), jnp.float32),
        pltpu.SemaphoreType.DMA,
        pltpu.SemaphoreType.DMA,
    ],
)
def _encode(idx_hbm, w2v_hbm, table_hbm, out_hbm, idx_v, emb_v, w2v_v, buf_v,
            gsem, wsem):
    wid = lax.axis_index("s") * NC + lax.axis_index("c")
    base = wid * BPW
    pltpu.sync_copy(idx_hbm.at[pl.ds(wid * NCH, NCH)], idx_v)
    for j in range(NCH):
        cbase = base + j * CHUNK
        wcopy = pltpu.async_copy(w2v_hbm.at[pl.ds(cbase, CHUNK)], w2v_v, wsem)

        def issue(g, carry):
            vec = idx_v[j, pl.ds(g * 16, 16)]
            for k in range(16):
                pltpu.make_async_copy(
                    table_hbm.at[pl.ds(vec[k], 1)],
                    emb_v.at[pl.ds(g * 16 + k, 1)],
                    gsem,
                ).start()
            return carry

        lax.fori_loop(0, CHUNK // 16, issue, 0)

        def drain(r, carry):
            pltpu.make_async_copy(
                table_hbm.at[pl.ds(0, 1)], emb_v.at[pl.ds(0, 1)], gsem
            ).wait()
            return carry

        lax.fori_loop(0, CHUNK, drain, 0)
        wcopy.wait()

        def body(r, carry):
            for c in range(EMB // 16):
                buf_v[r, pl.ds(c * 16, 16)] = emb_v[r, pl.ds(c * 16, 16)]
                buf_v[r, pl.ds(EMB + c * 16, 16)] = w2v_v[r, pl.ds(c * 16, 16)]
            return carry

        lax.fori_loop(0, CHUNK, body, 0)
        pltpu.sync_copy(buf_v, out_hbm.at[pl.ds(cbase, CHUNK)])


def kernel(category_id, word2vec, emb_table):
    idx = category_id.astype(jnp.int32).reshape(NW * NCH, CHUNK)
    return _encode(idx, word2vec, emb_table)
